# Initial kernel scaffold; baseline (speedup 1.0000x reference)
import jax
import jax.numpy as jnp
from jax.experimental import pallas as pl


def kernel(xyz, pair_i, pair_j, offsets):
    raise NotImplementedError("write your pallas kernel here")


# speedup vs baseline: 3.9807x; 3.9807x over previous
import functools

import jax
import jax.numpy as jnp
from jax import lax
from jax.experimental import pallas as pl
from jax.experimental.pallas import tpu as pltpu
from jax.experimental.pallas import tpu_sc as plsc

_L = 16
_PW = 8
_CH = 64


def _make_kernel(n_nodes, n_edges, block):
    info = plsc.get_sparse_core_info()
    nc, ns = info.num_cores, info.num_subcores
    nw = nc * ns
    assert n_edges % (nw * block) == 0, (n_edges, nw, block)
    assert block % _L == 0
    per_w = n_edges // nw
    nblk = per_w // block
    ngrp = block // _L
    nch = block // _CH

    mesh = plsc.VectorSubcoreMesh(core_axis_name="c", subcore_axis_name="s")

    @functools.partial(
        pl.kernel,
        out_type=(
            jax.ShapeDtypeStruct((3 * n_edges,), jnp.float32),
            jax.ShapeDtypeStruct((n_edges,), jnp.float32),
        ),
        mesh=mesh,
        scratch_types=[
            pltpu.VMEM_SHARED((n_nodes, _PW), jnp.float32),
            pltpu.VMEM((block,), jnp.int32),
            pltpu.VMEM((block,), jnp.int32),
            pltpu.VMEM((3 * block,), jnp.float32),
            pltpu.VMEM((block, _PW), jnp.float32),
            pltpu.VMEM((block, _PW), jnp.float32),
            pltpu.VMEM((3 * block,), jnp.float32),
            pltpu.VMEM((block,), jnp.float32),
            pltpu.SemaphoreType.DMA,
        ],
        compiler_params=pltpu.CompilerParams(
            needs_layout_passes=False, use_tc_tiling_on_sc=False
        ),
    )
    def run(xyz_hbm, pi_hbm, pj_hbm, off_hbm, diff_hbm, dist_hbm,
            xyz_sp, idx_i, idx_j, off_v, rows_i, rows_j, diff_v, dist_v, sem):
        c = lax.axis_index("c")
        s = lax.axis_index("s")
        wid = s * nc + c

        @pl.when(s == 0)
        def _stage():
            pltpu.sync_copy(xyz_hbm, xyz_sp)

        plsc.subcore_barrier()

        iota = lax.iota(jnp.int32, _L)
        czero = jnp.zeros((_L,), jnp.int32)
        cone = jnp.ones((_L,), jnp.int32)
        ctwo = cone + cone
        base_w = wid * per_w

        def blk_body(blk, carry):
            base = base_w + blk * block
            pltpu.sync_copy(pi_hbm.at[pl.ds(base, block)], idx_i)
            pltpu.sync_copy(pj_hbm.at[pl.ds(base, block)], idx_j)
            pltpu.sync_copy(off_hbm.at[pl.ds(3 * base, 3 * block)], off_v)
            cp_i = pltpu.async_copy(xyz_sp.at[idx_i], rows_i, sem)
            cp_j = pltpu.async_copy(xyz_sp.at[idx_j], rows_j, sem)
            cp_i.wait()
            cp_j.wait()

            def grp_body(g, carry2):
                e0 = g * _L
                ei = e0 + iota
                p3 = ei * 3
                p31 = p3 + cone
                p32 = p31 + cone
                xi = plsc.load_gather(rows_i, [ei, czero])
                yi = plsc.load_gather(rows_i, [ei, cone])
                zi = plsc.load_gather(rows_i, [ei, ctwo])
                xj = plsc.load_gather(rows_j, [ei, czero])
                yj = plsc.load_gather(rows_j, [ei, cone])
                zj = plsc.load_gather(rows_j, [ei, ctwo])
                ox = plsc.load_gather(off_v, [p3])
                oy = plsc.load_gather(off_v, [p31])
                oz = plsc.load_gather(off_v, [p32])
                dx = xj - xi + ox
                dy = yj - yi + oy
                dz = zj - zi + oz
                plsc.store_scatter(diff_v, [p3], dx)
                plsc.store_scatter(diff_v, [p31], dy)
                plsc.store_scatter(diff_v, [p32], dz)
                s2 = dx * dx + dy * dy + dz * dz
                ibits = plsc.bitcast(s2, jnp.int32)
                y = plsc.bitcast(jnp.int32(0x5F3759DF) - (ibits >> 1),
                                 jnp.float32)
                hs = s2 * 0.5
                y = y * (1.5 - hs * y * y)
                y = y * (1.5 - hs * y * y)
                y = y * (1.5 - hs * y * y)
                dist_v[pl.ds(e0, _L)] = s2 * y
                return carry2

            lax.fori_loop(0, ngrp, grp_body, 0)
            pltpu.sync_copy(diff_v, diff_hbm.at[pl.ds(3 * base, 3 * block)])
            pltpu.sync_copy(dist_v, dist_hbm.at[pl.ds(base, block)])
            return carry

        lax.fori_loop(0, nblk, blk_body, 0)

    return run


def kernel(xyz, pair_i, pair_j, offsets):
    n_nodes = xyz.shape[0]
    n_edges = pair_i.shape[0]
    run = _make_kernel(n_nodes, n_edges, block=1600)
    xyz_p = jnp.zeros((n_nodes, _PW), jnp.float32).at[:, :3].set(xyz)
    diff_flat, dist = run(
        xyz_p,
        pair_i.astype(jnp.int32),
        pair_j.astype(jnp.int32),
        offsets.reshape(-1),
    )
    return diff_flat.reshape(n_edges, 3), dist
